# baseline (speedup 1.0000x reference)
import functools
import jax
import jax.numpy as jnp
from jax.experimental import pallas as pl

_EPS = 1e-8


def _body(ea_ref, w_ref, bias_ref, ep_ref, out_ref):
    s = jnp.sum(ea_ref[0] * w_ref[...], axis=1) + bias_ref[0, 0]
    p = ep_ref[0]
    num = jax.lax.dot_general(
        p, s, (((2,), (0,)), ((), ())), preferred_element_type=jnp.float32
    )
    den = jnp.sum(p, axis=2)
    out_ref[0] = num / (den + _EPS)


def kernel(edge_attr, edge_paths, ptr, W, b):
    nB, nL, _, nE = edge_paths.shape
    nD = edge_attr.shape[1]
    TI = 128
    ea = edge_attr.reshape(nB, nE, nD)
    wr = W.reshape(1, nD)
    br = b.reshape(1, 1)
    grid = (nB, nL // TI)
    out = pl.pallas_call(
        _body,
        grid=grid,
        in_specs=[
            pl.BlockSpec((1, nE, nD), lambda bi, ic: (bi, 0, 0)),
            pl.BlockSpec((1, nD), lambda bi, ic: (0, 0)),
            pl.BlockSpec((1, 1), lambda bi, ic: (0, 0)),
            pl.BlockSpec((1, TI, nL, nE), lambda bi, ic: (bi, ic, 0, 0)),
        ],
        out_specs=pl.BlockSpec((1, TI, nL), lambda bi, ic: (bi, ic, 0)),
        out_shape=jax.ShapeDtypeStruct((nB, nL, nL), jnp.float32),
    )(ea, wr, br, edge_paths)
    return out


# speedup vs baseline: 1.1273x; 1.1273x over previous
import functools
import jax
import jax.numpy as jnp
from jax.experimental import pallas as pl

_EPS = 1e-8


def _body(ea_ref, w_ref, bias_ref, ep_ref, out_ref):
    s = jnp.sum(ea_ref[0] * w_ref[...], axis=1) + bias_ref[0, 0]
    p = ep_ref[0]
    num = jax.lax.dot_general(
        p, s, (((2,), (0,)), ((), ())), preferred_element_type=jnp.float32
    )
    den = jnp.sum(p, axis=2)
    out_ref[0] = num / (den + _EPS)


def kernel(edge_attr, edge_paths, ptr, W, b):
    nB, nL, _, nE = edge_paths.shape
    nD = edge_attr.shape[1]
    TI = 64
    ea = edge_attr.reshape(nB, nE, nD)
    wr = W.reshape(1, nD)
    br = b.reshape(1, 1)
    grid = (nB, nL // TI)
    out = pl.pallas_call(
        _body,
        grid=grid,
        in_specs=[
            pl.BlockSpec((1, nE, nD), lambda bi, ic: (bi, 0, 0)),
            pl.BlockSpec((1, nD), lambda bi, ic: (0, 0)),
            pl.BlockSpec((1, 1), lambda bi, ic: (0, 0)),
            pl.BlockSpec((1, TI, nL, nE), lambda bi, ic: (bi, ic, 0, 0)),
        ],
        out_specs=pl.BlockSpec((1, TI, nL), lambda bi, ic: (bi, ic, 0)),
        out_shape=jax.ShapeDtypeStruct((nB, nL, nL), jnp.float32),
    )(ea, wr, br, edge_paths)
    return out
